# baseline (speedup 1.0000x reference)
import jax
import jax.numpy as jnp
from jax import lax
from jax.experimental import pallas as pl
from jax.experimental.pallas import tpu as pltpu
from jax.experimental.pallas import tpu_sc as plsc

N_ROIS = 100000
N_REL = 3200000
SCALE = 28.0 / 1024.0

NC = 2
NS = 16
NW = NC * NS
CHUNK = 512
TPC = CHUNK // 128
SUB = 128
NSUB = CHUNK // SUB
NCHT = N_REL // CHUNK
MAXJ = (NCHT + NW - 1) // NW
NTILES = N_REL // 128


def _iota16():
  return lax.iota(jnp.int32, 16)


def _splat(v):
  return jnp.full((16,), v, jnp.int32)


def _sc_body(rois_hbm, si_hbm, oi_hbm, subjt_hbm, objt_hbm,
             si0, si1, oi0, oi1, sb0, sb1, ob0, ob1,
             so0, so1, oo0, oo1, semi0, semi1, semg0, semg1, semo0, semo1):
  si = (si0, si1)
  oi = (oi0, oi1)
  sb = (sb0, sb1)
  ob = (ob0, ob1)
  so = (so0, so1)
  oo = (oo0, oo1)
  semi = (semi0, semi1)
  semg = (semg0, semg1)
  semo = (semo0, semo1)

  wid = lax.axis_index("s") * NC + lax.axis_index("c")

  def cid(j):
    return j * NW + wid

  def valid(j):
    return cid(j) < NCHT

  def fire_inds(b, c):
    off = c * CHUNK
    for g in range(NSUB):
      pltpu.async_copy(si_hbm.at[pl.ds(off + g * SUB, SUB)], si[b].at[g],
                       semi[b])
      pltpu.async_copy(oi_hbm.at[pl.ds(off + g * SUB, SUB)], oi[b].at[g],
                       semi[b])

  def wait_inds(b):
    for g in range(NSUB):
      pltpu.make_async_copy(si_hbm.at[pl.ds(0, SUB)], si[b].at[g],
                            semi[b]).wait()
      pltpu.make_async_copy(oi_hbm.at[pl.ds(0, SUB)], oi[b].at[g],
                            semi[b]).wait()

  def fire_gathers(b):
    for g in range(NSUB):
      pltpu.async_copy(rois_hbm.at[si[b].at[g]],
                       sb[b].at[pl.ds(g * SUB, SUB), :], semg[b])
      pltpu.async_copy(rois_hbm.at[oi[b].at[g]],
                       ob[b].at[pl.ds(g * SUB, SUB), :], semg[b])

  def wait_gathers(b):
    pltpu.make_async_copy(rois_hbm.at[pl.ds(0, CHUNK), :], sb[b],
                          semg[b]).wait()
    pltpu.make_async_copy(rois_hbm.at[pl.ds(0, CHUNK), :], ob[b],
                          semg[b]).wait()

  def compute(b):
    k = jnp.full((16,), SCALE, jnp.float32)
    cols = [_splat(c) for c in range(5)]
    it = _iota16()
    for i in range(CHUNK // 16):
      lanes = it + _splat(i * 16)
      tile = _splat(i // 8)
      lane0 = it + _splat((i % 8) * 16)
      s0 = plsc.load_gather(sb[b], [lanes, cols[0]])
      s1 = plsc.load_gather(sb[b], [lanes, cols[1]])
      s2 = plsc.load_gather(sb[b], [lanes, cols[2]])
      s3 = plsc.load_gather(sb[b], [lanes, cols[3]])
      s4 = plsc.load_gather(sb[b], [lanes, cols[4]])
      o0 = plsc.load_gather(ob[b], [lanes, cols[0]])
      o1 = plsc.load_gather(ob[b], [lanes, cols[1]])
      o2 = plsc.load_gather(ob[b], [lanes, cols[2]])
      o3 = plsc.load_gather(ob[b], [lanes, cols[3]])
      o4 = plsc.load_gather(ob[b], [lanes, cols[4]])
      xmin = jnp.minimum(s1, o1)
      ymin = jnp.minimum(s2, o2)
      plsc.store_scatter(so[b], [cols[0], tile, lane0], s0)
      plsc.store_scatter(so[b], [cols[1], tile, lane0], (s1 - xmin) * k)
      plsc.store_scatter(so[b], [cols[2], tile, lane0], (s2 - ymin) * k)
      plsc.store_scatter(so[b], [cols[3], tile, lane0], (s3 - xmin) * k)
      plsc.store_scatter(so[b], [cols[4], tile, lane0], (s4 - ymin) * k)
      plsc.store_scatter(oo[b], [cols[0], tile, lane0], o0)
      plsc.store_scatter(oo[b], [cols[1], tile, lane0], (o1 - xmin) * k)
      plsc.store_scatter(oo[b], [cols[2], tile, lane0], (o2 - ymin) * k)
      plsc.store_scatter(oo[b], [cols[3], tile, lane0], (o3 - xmin) * k)
      plsc.store_scatter(oo[b], [cols[4], tile, lane0], (o4 - ymin) * k)

  def fire_out(b, c):
    t0 = c * TPC
    for col in range(5):
      pltpu.async_copy(so[b].at[col], subjt_hbm.at[pl.ds(t0, TPC), col, :],
                       semo[b])
      pltpu.async_copy(oo[b].at[col], objt_hbm.at[pl.ds(t0, TPC), col, :],
                       semo[b])

  def wait_out(b):
    for col in range(5):
      pltpu.make_async_copy(so[b].at[col],
                            subjt_hbm.at[pl.ds(0, TPC), col, :],
                            semo[b]).wait()
      pltpu.make_async_copy(oo[b].at[col],
                            objt_hbm.at[pl.ds(0, TPC), col, :],
                            semo[b]).wait()

  fire_inds(0, cid(jnp.int32(0)))
  wait_inds(0)
  fire_gathers(0)
  fire_inds(1, cid(jnp.int32(1)))

  def pair_body(j2, carry):
    for b in (0, 1):
      j = j2 * 2 + b
      nb = 1 - b

      @pl.when((j < MAXJ - 1) & valid(j + 1))
      def _prep():
        wait_inds(nb)
        fire_gathers(nb)

      @pl.when(valid(j))
      def _work():
        wait_gathers(b)

        @pl.when((j < MAXJ - 2) & valid(j + 2))
        def _pref():
          fire_inds(b, cid(j + 2))

        @pl.when(j >= 2)
        def _drain():
          wait_out(b)

        compute(b)
        fire_out(b, cid(j))
    return carry

  lax.fori_loop(0, MAXJ // 2, pair_body, jnp.int32(0))
  wait_out(0)
  wait_out(1)


@jax.jit
def kernel(rois, rel_inds):
  rois_pad = jnp.pad(rois, ((0, 0), (0, 11)))
  si_all = rel_inds[:, 1]
  oi_all = rel_inds[:, 2]
  mesh = plsc.VectorSubcoreMesh(core_axis_name="c", subcore_axis_name="s")
  f = pl.kernel(
      _sc_body,
      out_type=(
          jax.ShapeDtypeStruct((NTILES, 8, 128), jnp.float32),
          jax.ShapeDtypeStruct((NTILES, 8, 128), jnp.float32),
      ),
      mesh=mesh,
      compiler_params=pltpu.CompilerParams(
          needs_layout_passes=False, use_tc_tiling_on_sc=False),
      scratch_types=[
          pltpu.VMEM((NSUB, SUB), jnp.int32),
          pltpu.VMEM((NSUB, SUB), jnp.int32),
          pltpu.VMEM((NSUB, SUB), jnp.int32),
          pltpu.VMEM((NSUB, SUB), jnp.int32),
          pltpu.VMEM((CHUNK, 16), jnp.float32),
          pltpu.VMEM((CHUNK, 16), jnp.float32),
          pltpu.VMEM((CHUNK, 16), jnp.float32),
          pltpu.VMEM((CHUNK, 16), jnp.float32),
          pltpu.VMEM((5, TPC, 128), jnp.float32),
          pltpu.VMEM((5, TPC, 128), jnp.float32),
          pltpu.VMEM((5, TPC, 128), jnp.float32),
          pltpu.VMEM((5, TPC, 128), jnp.float32),
          pltpu.SemaphoreType.DMA,
          pltpu.SemaphoreType.DMA,
          pltpu.SemaphoreType.DMA,
          pltpu.SemaphoreType.DMA,
          pltpu.SemaphoreType.DMA,
          pltpu.SemaphoreType.DMA,
      ],
  )
  subj_t, obj_t = f(rois_pad, si_all, oi_all)
  subj = subj_t.transpose(0, 2, 1).reshape(N_REL, 8)[:, :5]
  obj = obj_t.transpose(0, 2, 1).reshape(N_REL, 8)[:, :5]
  return subj, obj


# speedup vs baseline: 1.0279x; 1.0279x over previous
import jax
import jax.numpy as jnp
from jax import lax
from jax.experimental import pallas as pl
from jax.experimental.pallas import tpu as pltpu
from jax.experimental.pallas import tpu_sc as plsc

N_ROIS = 100000
N_REL = 3200000
SCALE = 28.0 / 1024.0

NC = 2
NS = 16
NW = NC * NS
CHUNK = 512
TPC = CHUNK // 128
SUB = 128
NSUB = CHUNK // SUB
NCHT = N_REL // CHUNK
MAXJ = (NCHT + NW - 1) // NW
NTILES = N_REL // 128


def _iota16():
  return lax.iota(jnp.int32, 16)


def _splat(v):
  return jnp.full((16,), v, jnp.int32)


def _sc_body(rois_hbm, si_hbm, oi_hbm, subjt_hbm, objt_hbm,
             si0, si1, oi0, oi1, sb0, sb1, ob0, ob1,
             so0, so1, oo0, oo1, semi0, semi1, semg0, semg1, semo0, semo1):
  si = (si0, si1)
  oi = (oi0, oi1)
  sb = (sb0, sb1)
  ob = (ob0, ob1)
  so = (so0, so1)
  oo = (oo0, oo1)
  semi = (semi0, semi1)
  semg = (semg0, semg1)
  semo = (semo0, semo1)

  wid = lax.axis_index("s") * NC + lax.axis_index("c")

  def cid(j):
    return j * NW + wid

  def valid(j):
    return cid(j) < NCHT

  def fire_inds(b, c):
    off = c * CHUNK
    for g in range(NSUB):
      pltpu.async_copy(si_hbm.at[pl.ds(off + g * SUB, SUB)], si[b].at[g],
                       semi[b])
      pltpu.async_copy(oi_hbm.at[pl.ds(off + g * SUB, SUB)], oi[b].at[g],
                       semi[b])

  def wait_inds(b):
    for g in range(NSUB):
      pltpu.make_async_copy(si_hbm.at[pl.ds(0, SUB)], si[b].at[g],
                            semi[b]).wait()
      pltpu.make_async_copy(oi_hbm.at[pl.ds(0, SUB)], oi[b].at[g],
                            semi[b]).wait()

  def fire_gathers(b):
    for g in range(NSUB):
      pltpu.async_copy(rois_hbm.at[si[b].at[g]],
                       sb[b].at[pl.ds(g * SUB, SUB), :], semg[b])
      pltpu.async_copy(rois_hbm.at[oi[b].at[g]],
                       ob[b].at[pl.ds(g * SUB, SUB), :], semg[b])

  def wait_gathers(b):
    pltpu.make_async_copy(rois_hbm.at[pl.ds(0, CHUNK), :], sb[b],
                          semg[b]).wait()
    pltpu.make_async_copy(rois_hbm.at[pl.ds(0, CHUNK), :], ob[b],
                          semg[b]).wait()

  def compute(b):
    k = jnp.full((16,), SCALE, jnp.float32)
    cols = [_splat(c) for c in range(5)]
    it = _iota16()
    for i in range(CHUNK // 16):
      lanes = it + _splat(i * 16)
      tile = _splat(i // 8)
      lane0 = it + _splat((i % 8) * 16)
      s0 = plsc.load_gather(sb[b], [lanes, cols[0]])
      s1 = plsc.load_gather(sb[b], [lanes, cols[1]])
      s2 = plsc.load_gather(sb[b], [lanes, cols[2]])
      s3 = plsc.load_gather(sb[b], [lanes, cols[3]])
      s4 = plsc.load_gather(sb[b], [lanes, cols[4]])
      o0 = plsc.load_gather(ob[b], [lanes, cols[0]])
      o1 = plsc.load_gather(ob[b], [lanes, cols[1]])
      o2 = plsc.load_gather(ob[b], [lanes, cols[2]])
      o3 = plsc.load_gather(ob[b], [lanes, cols[3]])
      o4 = plsc.load_gather(ob[b], [lanes, cols[4]])
      xmin = jnp.minimum(s1, o1)
      ymin = jnp.minimum(s2, o2)
      plsc.store_scatter(so[b], [cols[0], tile, lane0], s0)
      plsc.store_scatter(so[b], [cols[1], tile, lane0], (s1 - xmin) * k)
      plsc.store_scatter(so[b], [cols[2], tile, lane0], (s2 - ymin) * k)
      plsc.store_scatter(so[b], [cols[3], tile, lane0], (s3 - xmin) * k)
      plsc.store_scatter(so[b], [cols[4], tile, lane0], (s4 - ymin) * k)
      plsc.store_scatter(oo[b], [cols[0], tile, lane0], o0)
      plsc.store_scatter(oo[b], [cols[1], tile, lane0], (o1 - xmin) * k)
      plsc.store_scatter(oo[b], [cols[2], tile, lane0], (o2 - ymin) * k)
      plsc.store_scatter(oo[b], [cols[3], tile, lane0], (o3 - xmin) * k)
      plsc.store_scatter(oo[b], [cols[4], tile, lane0], (o4 - ymin) * k)

  def fire_out(b, c):
    t0 = c * TPC
    for col in range(5):
      pltpu.async_copy(so[b].at[col], subjt_hbm.at[pl.ds(t0, TPC), col, :],
                       semo[b])
      pltpu.async_copy(oo[b].at[col], objt_hbm.at[pl.ds(t0, TPC), col, :],
                       semo[b])

  def wait_out(b):
    for col in range(5):
      pltpu.make_async_copy(so[b].at[col],
                            subjt_hbm.at[pl.ds(0, TPC), col, :],
                            semo[b]).wait()
      pltpu.make_async_copy(oo[b].at[col],
                            objt_hbm.at[pl.ds(0, TPC), col, :],
                            semo[b]).wait()

  fire_inds(0, cid(jnp.int32(0)))
  wait_inds(0)
  fire_gathers(0)
  fire_inds(1, cid(jnp.int32(1)))

  def pair_body(j2, carry):
    for b in (0, 1):
      j = j2 * 2 + b
      nb = 1 - b

      @pl.when((j < MAXJ - 1) & valid(j + 1))
      def _prep():
        wait_inds(nb)
        fire_gathers(nb)

      @pl.when(valid(j))
      def _work():
        wait_gathers(b)

        @pl.when((j < MAXJ - 2) & valid(j + 2))
        def _pref():
          fire_inds(b, cid(j + 2))

        @pl.when(j >= 2)
        def _drain():
          wait_out(b)

        compute(b)
        fire_out(b, cid(j))
    return carry

  lax.fori_loop(0, MAXJ // 2, pair_body, jnp.int32(0))
  wait_out(0)
  wait_out(1)


@jax.jit
def kernel(rois, rel_inds):
  rois_pad = jnp.pad(rois, ((0, 0), (0, 3)))
  si_all = rel_inds[:, 1]
  oi_all = rel_inds[:, 2]
  mesh = plsc.VectorSubcoreMesh(core_axis_name="c", subcore_axis_name="s")
  f = pl.kernel(
      _sc_body,
      out_type=(
          jax.ShapeDtypeStruct((NTILES, 8, 128), jnp.float32),
          jax.ShapeDtypeStruct((NTILES, 8, 128), jnp.float32),
      ),
      mesh=mesh,
      compiler_params=pltpu.CompilerParams(
          needs_layout_passes=False, use_tc_tiling_on_sc=False),
      scratch_types=[
          pltpu.VMEM((NSUB, SUB), jnp.int32),
          pltpu.VMEM((NSUB, SUB), jnp.int32),
          pltpu.VMEM((NSUB, SUB), jnp.int32),
          pltpu.VMEM((NSUB, SUB), jnp.int32),
          pltpu.VMEM((CHUNK, 8), jnp.float32),
          pltpu.VMEM((CHUNK, 8), jnp.float32),
          pltpu.VMEM((CHUNK, 8), jnp.float32),
          pltpu.VMEM((CHUNK, 8), jnp.float32),
          pltpu.VMEM((5, TPC, 128), jnp.float32),
          pltpu.VMEM((5, TPC, 128), jnp.float32),
          pltpu.VMEM((5, TPC, 128), jnp.float32),
          pltpu.VMEM((5, TPC, 128), jnp.float32),
          pltpu.SemaphoreType.DMA,
          pltpu.SemaphoreType.DMA,
          pltpu.SemaphoreType.DMA,
          pltpu.SemaphoreType.DMA,
          pltpu.SemaphoreType.DMA,
          pltpu.SemaphoreType.DMA,
      ],
  )
  subj_t, obj_t = f(rois_pad, si_all, oi_all)
  subj = subj_t.transpose(0, 2, 1).reshape(N_REL, 8)[:, :5]
  obj = obj_t.transpose(0, 2, 1).reshape(N_REL, 8)[:, :5]
  return subj, obj
